# baseline (speedup 1.0000x reference)
import functools

import jax
import jax.numpy as jnp
from jax import lax
from jax.experimental import pallas as pl
from jax.experimental.pallas import tpu as pltpu
from jax.experimental.pallas import tpu_sc as plsc

NUM_CLASSES = 1000
EPS = 1e-7

ROWS, COLS = 16384, 512
N = ROWS * COLS
NC, NS, L = 2, 16, 16
NW = NC * NS
PER_W = N // NW
B = 1024

CH1 = 32768
NCH1 = PER_W // CH1
CH2 = 16384
NCH2 = PER_W // CH2

_mesh = plsc.VectorSubcoreMesh(core_axis_name="c", subcore_axis_name="s")
_params = pltpu.CompilerParams(needs_layout_passes=False)


def _lane_iota():
    return lax.iota(jnp.int32, L)


@functools.partial(
    pl.kernel,
    out_type=jax.ShapeDtypeStruct((NW * B,), jnp.int32),
    mesh=_mesh,
    scratch_types=[
        pltpu.VMEM((CH1,), jnp.int32),
        pltpu.VMEM((CH1,), jnp.int32),
        pltpu.VMEM((B * L,), jnp.int32),
        pltpu.VMEM((B,), jnp.int32),
        pltpu.SemaphoreType.DMA,
        pltpu.SemaphoreType.DMA,
    ],
    compiler_params=_params,
)
def _hist_kernel(x_hbm, out_hbm, buf_a, buf_b, hist, counts, sem_a, sem_b):
    wid = lax.axis_index("s") * NC + lax.axis_index("c")
    base = wid * PER_W
    lanes = _lane_iota()
    zeros = jnp.zeros((L,), jnp.int32)
    ones = jnp.ones((L,), jnp.int32)

    bufs = (buf_a, buf_b)
    sems = (sem_a, sem_b)
    copies = [
        pltpu.async_copy(x_hbm.at[pl.ds(base, CH1)], buf_a, sem_a),
        None,
    ]

    @plsc.parallel_loop(0, B, unroll=8)
    def _zero(i):
        hist[pl.ds(i * L, L)] = zeros
    for c in range(NCH1):
        if c + 1 < NCH1:
            nxt = (c + 1) % 2
            copies[nxt] = pltpu.async_copy(
                x_hbm.at[pl.ds(base + (c + 1) * CH1, CH1)], bufs[nxt], sems[nxt]
            )
        copies[c % 2].wait()
        cur = bufs[c % 2]

        @plsc.parallel_loop(0, CH1 // L, unroll=16)
        def _groups(g):
            idx = cur[pl.ds(g * L, L)]
            addr = idx * L + lanes
            plsc.addupdate_scatter(hist, [addr], ones)

    @plsc.parallel_loop(0, B // L, unroll=2)
    def _reduce(grp):
        b0 = grp * L
        acc = zeros
        for d in range(L):
            slot = lax.rem(lanes + d, L)
            acc = acc + plsc.load_gather(hist, [(b0 + lanes) * L + slot])
        counts[pl.ds(b0, L)] = acc

    pltpu.sync_copy(counts, out_hbm.at[pl.ds(wid * B, B)])


@functools.partial(
    pl.kernel,
    out_type=jax.ShapeDtypeStruct((N,), jnp.float32),
    mesh=_mesh,
    scratch_types=[
        pltpu.VMEM((CH2,), jnp.int32),
        pltpu.VMEM((CH2,), jnp.int32),
        pltpu.VMEM((CH2,), jnp.float32),
        pltpu.VMEM((CH2,), jnp.float32),
        pltpu.VMEM((NW * B,), jnp.int32),
        pltpu.VMEM((B * L,), jnp.float32),
        pltpu.SemaphoreType.DMA,
        pltpu.SemaphoreType.DMA,
        pltpu.SemaphoreType.DMA,
        pltpu.SemaphoreType.DMA,
    ],
    compiler_params=_params,
)
def _gather_kernel(x_hbm, parts_hbm, out_hbm, ib_a, ib_b, ob_a, ob_b,
                   parts, inv_rep, isem_a, isem_b, osem_a, osem_b):
    wid = lax.axis_index("s") * NC + lax.axis_index("c")
    base = wid * PER_W
    lanes = _lane_iota()

    ibufs = (ib_a, ib_b)
    obufs = (ob_a, ob_b)
    isems = (isem_a, isem_b)
    osems = (osem_a, osem_b)
    in_copies = [
        pltpu.async_copy(x_hbm.at[pl.ds(base, CH2)], ib_a, isem_a),
        pltpu.async_copy(x_hbm.at[pl.ds(base + CH2, CH2)], ib_b, isem_b),
    ]

    pltpu.sync_copy(parts_hbm, parts)

    @plsc.parallel_loop(0, B // L, unroll=2)
    def _build(grp):
        b0 = grp * L
        acc = jnp.zeros((L,), jnp.int32)
        for w in range(NW):
            acc = acc + parts[pl.ds(w * B + b0, L)]
        inv = 1.0 / jnp.maximum(acc.astype(jnp.float32), EPS)
        for d in range(L):
            slot = lax.rem(lanes + d, L)
            plsc.store_scatter(inv_rep, [(b0 + lanes) * L + slot], inv)

    out_copies = [None, None]
    for c in range(NCH2):
        p = c % 2
        in_copies[p].wait()
        if out_copies[p] is not None:
            out_copies[p].wait()
        cur_i, cur_o = ibufs[p], obufs[p]

        @plsc.parallel_loop(0, CH2 // 128, unroll=2)
        def _rowrun(rg):
            sbase = rg * 128
            dbase = (
                lax.shift_right_logical(rg, 5) * 4096
                + jnp.bitwise_and(rg, 7) * 512
                + jnp.bitwise_and(lax.shift_right_logical(rg, 3), 3) * 128
            )
            for k in range(8):
                idx = cur_i[pl.ds(sbase + k * L, L)]
                vals = plsc.load_gather(inv_rep, [idx * L + lanes])
                cur_o[pl.ds(dbase + k * L, L)] = vals

        out_copies[p] = pltpu.async_copy(
            cur_o, out_hbm.at[pl.ds(base + c * CH2, CH2)], osems[p]
        )
        if c + 2 < NCH2:
            in_copies[p] = pltpu.async_copy(
                x_hbm.at[pl.ds(base + (c + 2) * CH2, CH2)], ibufs[p], isems[p]
            )
    for oc in out_copies:
        if oc is not None:
            oc.wait()


def kernel(inputs):
    x4 = jnp.reshape(inputs.astype(jnp.int32), (ROWS // 8, 8, COLS // 128, 128))
    flat = jnp.reshape(jnp.transpose(x4, (0, 2, 1, 3)), (N,))
    partials = _hist_kernel(flat)
    out = _gather_kernel(flat, partials)
    return jnp.expand_dims(out, axis=-1)


# speedup vs baseline: 1.0116x; 1.0116x over previous
import functools

import jax
import jax.numpy as jnp
from jax import lax
from jax.experimental import pallas as pl
from jax.experimental.pallas import tpu as pltpu
from jax.experimental.pallas import tpu_sc as plsc

NUM_CLASSES = 1000
EPS = 1e-7

ROWS, COLS = 16384, 512
N = ROWS * COLS
NC, NS, L = 2, 16, 16
NW = NC * NS
PER_W = N // NW
B = 1024

CH1 = 32768
NCH1 = PER_W // CH1
CH2 = 16384
NCH2 = PER_W // CH2

_mesh = plsc.VectorSubcoreMesh(core_axis_name="c", subcore_axis_name="s")
_params = pltpu.CompilerParams(needs_layout_passes=False)


def _lane_iota():
    return lax.iota(jnp.int32, L)


@functools.partial(
    pl.kernel,
    out_type=jax.ShapeDtypeStruct((NW * B,), jnp.int32),
    mesh=_mesh,
    scratch_types=[
        pltpu.VMEM((CH1,), jnp.int32),
        pltpu.VMEM((CH1,), jnp.int32),
        pltpu.VMEM((B * L,), jnp.int32),
        pltpu.VMEM((B,), jnp.int32),
        pltpu.SemaphoreType.DMA,
        pltpu.SemaphoreType.DMA,
    ],
    compiler_params=_params,
)
def _hist_kernel(x_hbm, out_hbm, buf_a, buf_b, hist, counts, sem_a, sem_b):
    wid = lax.axis_index("s") * NC + lax.axis_index("c")
    base = wid * PER_W
    lanes = _lane_iota()
    zeros = jnp.zeros((L,), jnp.int32)
    ones = jnp.ones((L,), jnp.int32)

    bufs = (buf_a, buf_b)
    sems = (sem_a, sem_b)
    copies = [
        pltpu.async_copy(x_hbm.at[pl.ds(base, CH1)], buf_a, sem_a),
        None,
    ]

    @pl.loop(0, B, unroll=8)
    def _zero(i):
        hist[pl.ds(i * L, L)] = zeros
    for c in range(NCH1):
        if c + 1 < NCH1:
            nxt = (c + 1) % 2
            copies[nxt] = pltpu.async_copy(
                x_hbm.at[pl.ds(base + (c + 1) * CH1, CH1)], bufs[nxt], sems[nxt]
            )
        copies[c % 2].wait()
        cur = bufs[c % 2]

        @plsc.parallel_loop(0, CH1 // L, unroll=16)
        def _groups(g):
            idx = cur[pl.ds(g * L, L)]
            addr = idx * L + lanes
            plsc.addupdate_scatter(hist, [addr], ones)

    @pl.loop(0, B // L)
    def _reduce(grp):
        b0 = grp * L
        acc = zeros
        for d in range(L):
            slot = lax.rem(lanes + d, L)
            acc = acc + plsc.load_gather(hist, [(b0 + lanes) * L + slot])
        counts[pl.ds(b0, L)] = acc

    pltpu.sync_copy(counts, out_hbm.at[pl.ds(wid * B, B)])


@functools.partial(
    pl.kernel,
    out_type=jax.ShapeDtypeStruct((N,), jnp.float32),
    mesh=_mesh,
    scratch_types=[
        pltpu.VMEM((CH2,), jnp.int32),
        pltpu.VMEM((CH2,), jnp.int32),
        pltpu.VMEM((CH2,), jnp.float32),
        pltpu.VMEM((CH2,), jnp.float32),
        pltpu.VMEM((NW * B,), jnp.int32),
        pltpu.VMEM((B * L,), jnp.float32),
        pltpu.SemaphoreType.DMA,
        pltpu.SemaphoreType.DMA,
        pltpu.SemaphoreType.DMA,
        pltpu.SemaphoreType.DMA,
    ],
    compiler_params=_params,
)
def _gather_kernel(x_hbm, parts_hbm, out_hbm, ib_a, ib_b, ob_a, ob_b,
                   parts, inv_rep, isem_a, isem_b, osem_a, osem_b):
    wid = lax.axis_index("s") * NC + lax.axis_index("c")
    base = wid * PER_W
    lanes = _lane_iota()

    ibufs = (ib_a, ib_b)
    obufs = (ob_a, ob_b)
    isems = (isem_a, isem_b)
    osems = (osem_a, osem_b)
    in_copies = [
        pltpu.async_copy(x_hbm.at[pl.ds(base, CH2)], ib_a, isem_a),
        pltpu.async_copy(x_hbm.at[pl.ds(base + CH2, CH2)], ib_b, isem_b),
    ]

    pltpu.sync_copy(parts_hbm, parts)

    @pl.loop(0, B // L)
    def _build(grp):
        b0 = grp * L
        acc = jnp.zeros((L,), jnp.int32)
        for w in range(NW):
            acc = acc + parts[pl.ds(w * B + b0, L)]
        inv = 1.0 / jnp.maximum(acc.astype(jnp.float32), EPS)
        for d in range(L):
            slot = lax.rem(lanes + d, L)
            plsc.store_scatter(inv_rep, [(b0 + lanes) * L + slot], inv)

    out_copies = [None, None]
    for c in range(NCH2):
        p = c % 2
        in_copies[p].wait()
        if out_copies[p] is not None:
            out_copies[p].wait()
        cur_i, cur_o = ibufs[p], obufs[p]

        @plsc.parallel_loop(0, CH2 // 128, unroll=2)
        def _rowrun(rg):
            sbase = rg * 128
            dbase = (
                lax.shift_right_logical(rg, 5) * 4096
                + jnp.bitwise_and(rg, 7) * 512
                + jnp.bitwise_and(lax.shift_right_logical(rg, 3), 3) * 128
            )
            for k in range(8):
                idx = cur_i[pl.ds(sbase + k * L, L)]
                vals = plsc.load_gather(inv_rep, [idx * L + lanes])
                cur_o[pl.ds(dbase + k * L, L)] = vals

        out_copies[p] = pltpu.async_copy(
            cur_o, out_hbm.at[pl.ds(base + c * CH2, CH2)], osems[p]
        )
        if c + 2 < NCH2:
            in_copies[p] = pltpu.async_copy(
                x_hbm.at[pl.ds(base + (c + 2) * CH2, CH2)], ibufs[p], isems[p]
            )
    for oc in out_copies:
        if oc is not None:
            oc.wait()


def kernel(inputs):
    x4 = jnp.reshape(inputs.astype(jnp.int32), (ROWS // 8, 8, COLS // 128, 128))
    flat = jnp.reshape(jnp.transpose(x4, (0, 2, 1, 3)), (N,))
    partials = _hist_kernel(flat)
    out = _gather_kernel(flat, partials)
    return jnp.expand_dims(out, axis=-1)
